# baseline (speedup 1.0000x reference)
import functools

import jax
import jax.numpy as jnp
from jax import lax
from jax.experimental import pallas as pl
from jax.experimental.pallas import tpu as pltpu
from jax.experimental.pallas import tpu_sc as plsc

_INFO = plsc.get_sparse_core_info()
_NC = _INFO.num_cores
_NS = _INFO.num_subcores
_NW = _NC * _NS
_LANES = _INFO.num_lanes
_G = 2
_PR = 128
_PD = 128
_NBUF = 4


@functools.lru_cache(maxsize=None)
def _make_embedding_bag(B, H, D, V):
    assert B % (_NW * _G) == 0
    S = B // (_NW * _G)
    R = _G * H
    RF = -(-R // 8) * 8
    assert RF <= _PR
    KD = D // _LANES

    mesh = plsc.VectorSubcoreMesh(core_axis_name="c", subcore_axis_name="s")

    @functools.partial(
        pl.kernel,
        mesh=mesh,
        compiler_params=pltpu.CompilerParams(
            use_tc_tiling_on_sc=False, needs_layout_passes=False),
        out_type=jax.ShapeDtypeStruct((B // _G, _G * D), jnp.float32),
        scratch_types=[
            pltpu.VMEM((S, _PR), jnp.int32),
            pltpu.VMEM((S, _PR), jnp.float32),
            [pltpu.VMEM((RF, D), jnp.bfloat16)] * _NBUF,
            pltpu.VMEM((S, _G * D), jnp.float32),
            [pltpu.SemaphoreType.DMA] * _NBUF,
        ],
    )
    def bag(table_hbm, idx_hbm, wgt_hbm, out_hbm,
            idx_v, wgt_v, rows_bufs, out_v, sems):
        wid = lax.axis_index("s") * _NC + lax.axis_index("c")
        pltpu.sync_copy(idx_hbm.at[wid], idx_v)
        pltpu.sync_copy(wgt_hbm.at[wid], wgt_v)

        def gcopy(s, b):
            return pltpu.make_async_copy(
                table_hbm.at[idx_v.at[s, pl.ds(0, RF)]],
                rows_bufs[b], sems[b])

        def compute(s, rows_v):
            for j in range(_G):
                base = j * H
                chunk_offs = [0, 16, 32, H - _LANES]
                wvecs = [wgt_v[s, pl.ds(base + o, _LANES)] for o in chunk_offs]

                def wlane(n):
                    if n < 48:
                        return wvecs[n // 16][n % 16]
                    return wvecs[3][n - (H - _LANES)]

                def row_vecs(p):
                    out = []
                    for c in range(D // 32):
                        v = rows_v[p, pl.ds(c * 32, 32)]
                        a, b = plsc.unpack(v, format=plsc.PackFormat.INTERLEAVED)
                        out += [a, b]
                    return out

                w0 = wlane(0)
                acc = [v * w0 for v in row_vecs(j * H)]
                for n in range(1, H):
                    p = j * H + n
                    w = wlane(n)
                    rv = row_vecs(p)
                    for k in range(KD):
                        acc[k] = acc[k] + rv[k] * w

                for k in range(KD):
                    out_v[s, pl.ds(j * D + k * _LANES, _LANES)] = acc[k]

        for i in range(_NBUF - 1):
            gcopy(i, i).start()

        def round_(t, carry):
            s0 = t * _NBUF
            for b in range(_NBUF):
                s = s0 + b
                gcopy(s, b).wait()
                compute(s, rows_bufs[b])
                nxt = s + _NBUF - 1

                @pl.when(nxt < S)
                def _():
                    gcopy(nxt, (b + _NBUF - 1) % _NBUF).start()
            return carry

        lax.fori_loop(0, S // _NBUF, round_, 0)
        pltpu.sync_copy(out_v, out_hbm.at[pl.ds(wid * S, S)])

    return bag


def kernel(X, X_w, W):
    B, H = X.shape
    V, D = W.shape
    S = B // (_NW * _G)
    R = _G * H
    W16 = W[:V - 1].astype(jnp.bfloat16)
    Wt = jnp.pad(W16, ((0, 0), (0, _PD - D))).reshape(2 * (V - 1), D)
    Xr = X.astype(jnp.int32).reshape(_NW, S, R) * 2
    spread = 2 * (((jnp.arange(S)[:, None] * (_PR - R)
                    + jnp.arange(_PR - R)[None, :]) * 997) % (V - 1))
    spread = jnp.broadcast_to(spread[None].astype(jnp.int32), (_NW, S, _PR - R))
    Xp = jnp.concatenate([Xr, spread], axis=2)
    Wr = X_w.astype(jnp.float32).reshape(_NW, S, R)
    Wp = jnp.pad(Wr, ((0, 0), (0, 0), (0, _PR - R)))
    out2 = _make_embedding_bag(B, H, D, V)(Wt, Xp, Wp)
    perm = [0] * D
    for q in range(D // 32):
        for i in range(16):
            perm[32 * q + 2 * i] = 32 * q + i
            perm[32 * q + 2 * i + 1] = 32 * q + 16 + i
    return out2.reshape(B, D)[:, jnp.array(perm)]


# speedup vs baseline: 2.0045x; 2.0045x over previous
import functools

import jax
import jax.numpy as jnp
from jax import lax
from jax.experimental import pallas as pl
from jax.experimental.pallas import tpu as pltpu
from jax.experimental.pallas import tpu_sc as plsc

_INFO = plsc.get_sparse_core_info()
_NC = _INFO.num_cores
_NS = _INFO.num_subcores
_NW = _NC * _NS
_LANES = _INFO.num_lanes
_G = 2
_PR = 128
_PD = 128
_NBUF = 4


@functools.lru_cache(maxsize=None)
def _make_embedding_bag(B, H, D, V):
    assert B % (_NW * _G) == 0
    S = B // (_NW * _G)
    R = _G * H
    RF = -(-R // 8) * 8
    assert RF <= _PR
    KD = D // _LANES

    mesh = plsc.VectorSubcoreMesh(core_axis_name="c", subcore_axis_name="s")

    @functools.partial(
        pl.kernel,
        mesh=mesh,
        compiler_params=pltpu.CompilerParams(use_tc_tiling_on_sc=False),
        out_type=jax.ShapeDtypeStruct((B // _G, _G * D), jnp.float32),
        scratch_types=[
            pltpu.VMEM((S, _PR), jnp.int32),
            pltpu.VMEM((S, _PR), jnp.float32),
            [pltpu.VMEM((RF, D), jnp.float32)] * _NBUF,
            pltpu.VMEM((S, _G * D), jnp.float32),
            [pltpu.SemaphoreType.DMA] * _NBUF,
        ],
    )
    def bag(table_hbm, idx_hbm, wgt_hbm, out_hbm,
            idx_v, wgt_v, rows_bufs, out_v, sems):
        wid = lax.axis_index("s") * _NC + lax.axis_index("c")
        pltpu.sync_copy(idx_hbm.at[wid], idx_v)
        pltpu.sync_copy(wgt_hbm.at[wid], wgt_v)

        def gcopy(s, b):
            return pltpu.make_async_copy(
                table_hbm.at[idx_v.at[s, pl.ds(0, RF)]],
                rows_bufs[b], sems[b])

        def compute(s, rows_v):
            for j in range(_G):
                base = j * H
                chunk_offs = [0, 16, 32, H - _LANES]
                wvecs = [wgt_v[s, pl.ds(base + o, _LANES)] for o in chunk_offs]

                def wlane(n):
                    if n < 48:
                        return wvecs[n // 16][n % 16]
                    return wvecs[3][n - (H - _LANES)]

                acc = [rows_v[j * H, pl.ds(k * _LANES, _LANES)] * wlane(0)
                       for k in range(KD)]
                for n in range(1, H):
                    p = j * H + n
                    w = wlane(n)
                    for k in range(KD):
                        acc[k] = acc[k] + rows_v[p, pl.ds(k * _LANES, _LANES)] * w

                for k in range(KD):
                    out_v[s, pl.ds(j * D + k * _LANES, _LANES)] = acc[k]

        for i in range(_NBUF - 1):
            gcopy(i, i).start()

        def round_(t, carry):
            s0 = t * _NBUF
            for b in range(_NBUF):
                s = s0 + b
                gcopy(s, b).wait()
                compute(s, rows_bufs[b])
                nxt = s + _NBUF - 1

                @pl.when(nxt < S)
                def _():
                    gcopy(nxt, (b + _NBUF - 1) % _NBUF).start()
            return carry

        lax.fori_loop(0, S // _NBUF, round_, 0)
        pltpu.sync_copy(out_v, out_hbm.at[pl.ds(wid * S, S)])

    return bag


def kernel(X, X_w, W):
    B, H = X.shape
    V, D = W.shape
    S = B // (_NW * _G)
    R = _G * H
    Wt = jnp.pad(W[:V - 1], ((0, 0), (0, _PD - D))).reshape(2 * (V - 1), D)
    Xr = X.astype(jnp.int32).reshape(_NW, S, R) * 2
    spread = 2 * (((jnp.arange(S)[:, None] * (_PR - R)
                    + jnp.arange(_PR - R)[None, :]) * 997) % (V - 1))
    spread = jnp.broadcast_to(spread[None].astype(jnp.int32), (_NW, S, _PR - R))
    Xp = jnp.concatenate([Xr, spread], axis=2)
    Wr = X_w.astype(jnp.float32).reshape(_NW, S, R)
    Wp = jnp.pad(Wr, ((0, 0), (0, 0), (0, _PR - R)))
    out2 = _make_embedding_bag(B, H, D, V)(Wt, Xp, Wp)
    return out2.reshape(B, D)
